# Initial kernel scaffold; baseline (speedup 1.0000x reference)
import jax
import jax.numpy as jnp
from jax.experimental import pallas as pl


def kernel(x, Wg, bg, W_gate, W_up, W_down):
    raise NotImplementedError("write your pallas kernel here")


# speedup vs baseline: 6.1387x; 6.1387x over previous
import functools

import jax
import jax.numpy as jnp
from jax import lax
from jax.experimental import pallas as pl
from jax.experimental.pallas import tpu as pltpu
from jax.experimental.pallas import tpu_sc as plsc

E = 64
D = 1024
F = 1024
N = 2048
TM = 128
NT = N // TM
G = NT + E


def _routing_body(x_ref, wg_ref, bg_ref, out_ref):
    logits = jnp.dot(x_ref[...], wg_ref[...],
                     preferred_element_type=jnp.float32) + bg_ref[...]
    out_ref[...] = jnp.argmax(logits, axis=-1).astype(jnp.int32)


def _route(x, Wg, bg):
    return pl.pallas_call(
        _routing_body,
        grid=(8,),
        in_specs=[
            pl.BlockSpec((N // 8, D), lambda i: (i, 0)),
            pl.BlockSpec((D, E), lambda i: (0, 0)),
            pl.BlockSpec((1, E), lambda i: (0, 0)),
        ],
        out_specs=pl.BlockSpec((N // 8,), lambda i: (i,)),
        out_shape=jax.ShapeDtypeStruct((N,), jnp.int32),
    )(x, Wg, bg.reshape(1, E))


def _sc_gather_rows(table, idx):
    info = plsc.get_sparse_core_info()
    nw = info.num_cores * info.num_subcores
    b_per_w = N // nw
    mesh = plsc.VectorSubcoreMesh(core_axis_name="c", subcore_axis_name="s")

    @functools.partial(
        pl.kernel,
        out_type=jax.ShapeDtypeStruct((N, D), jnp.float32),
        mesh=mesh,
        scratch_types=[
            pltpu.VMEM((b_per_w,), jnp.int32),
            pltpu.VMEM((b_per_w, D), jnp.float32),
            pltpu.SemaphoreType.DMA,
        ],
    )
    def gather_kernel(table_hbm, idx_hbm, out_hbm, idx_v, rows_v, sem):
        wid = lax.axis_index("s") * info.num_cores + lax.axis_index("c")
        base = wid * b_per_w
        pltpu.sync_copy(idx_hbm.at[pl.ds(base, b_per_w)], idx_v)
        pltpu.async_copy(table_hbm.at[idx_v], rows_v, sem).wait()
        pltpu.sync_copy(rows_v, out_hbm.at[pl.ds(base, b_per_w)])

    return gather_kernel(table, idx)


def _gmm_body(meta_ref, xs_ref, wg_ref, wu_ref, wd_ref, out_ref):
    g = pl.program_id(0)
    tile = meta_ref[1, g]
    start = meta_ref[2, g]
    end = meta_ref[3, g]

    xb = xs_ref[...]
    a = jnp.dot(xb, wg_ref[0], preferred_element_type=jnp.float32)
    b = jnp.dot(xb, wu_ref[0], preferred_element_type=jnp.float32)
    h = a * jax.nn.sigmoid(a) * b
    y = jnp.dot(h, wd_ref[0], preferred_element_type=jnp.float32)

    rows = tile * TM + lax.broadcasted_iota(jnp.int32, (TM, 1), 0)
    mask = (rows >= start) & (rows < end)
    out_ref[...] = jnp.where(mask, y, out_ref[...])


def _gmm(xs, W_gate, W_up, W_down, meta):
    grid_spec = pltpu.PrefetchScalarGridSpec(
        num_scalar_prefetch=1,
        grid=(G,),
        in_specs=[
            pl.BlockSpec((TM, D), lambda g, m: (m[1, g], 0)),
            pl.BlockSpec((1, D, F), lambda g, m: (m[0, g], 0, 0)),
            pl.BlockSpec((1, D, F), lambda g, m: (m[0, g], 0, 0)),
            pl.BlockSpec((1, F, D), lambda g, m: (m[0, g], 0, 0)),
        ],
        out_specs=pl.BlockSpec((TM, D), lambda g, m: (m[1, g], 0)),
    )
    return pl.pallas_call(
        _gmm_body,
        grid_spec=grid_spec,
        out_shape=jax.ShapeDtypeStruct((N, D), jnp.float32),
    )(meta, xs, W_gate, W_up, W_down)


def _work_metadata(top_i):
    counts = jnp.bincount(top_i, length=E)
    offsets = jnp.concatenate([jnp.zeros((1,), jnp.int32),
                               jnp.cumsum(counts).astype(jnp.int32)])
    start_e = offsets[:E]
    end_e = offsets[1:]
    t0 = start_e // TM
    t1 = jnp.where(counts > 0, (end_e - 1) // TM, t0 - 1)
    w_e = jnp.maximum(t1 - t0 + 1, 0)
    cw = jnp.cumsum(w_e)
    gids = jnp.arange(G, dtype=jnp.int32)
    eg = jnp.minimum(jnp.searchsorted(cw, gids, side="right"), E - 1)
    prev = jnp.where(eg > 0, cw[jnp.maximum(eg - 1, 0)], 0)
    tile = t0[eg] + (gids - prev)
    valid = gids < cw[E - 1]
    last_e = jnp.max(jnp.where(counts > 0, jnp.arange(E), 0))
    expert_g = jnp.where(valid, eg, last_e)
    tile_g = jnp.where(valid, tile, NT - 1)
    start_g = jnp.where(valid, start_e[eg], 0)
    end_g = jnp.where(valid, end_e[eg], 0)
    return jnp.stack([expert_g, tile_g, start_g, end_g]).astype(jnp.int32)


def kernel(x, Wg, bg, W_gate, W_up, W_down):
    top_i = _route(x, Wg, bg)

    sort_idx = jnp.argsort(top_i, stable=True).astype(jnp.int32)
    inv_idx = jnp.zeros((N,), jnp.int32).at[sort_idx].set(
        jnp.arange(N, dtype=jnp.int32))
    meta = _work_metadata(top_i)

    xs = _sc_gather_rows(x, sort_idx)
    ys = _gmm(xs, W_gate, W_up, W_down, meta)
    return _sc_gather_rows(ys, inv_idx)
